# Initial kernel scaffold; baseline (speedup 1.0000x reference)
import jax
import jax.numpy as jnp
from jax.experimental import pallas as pl


def kernel(player_features, hand_tab, suit_tab, bid_tab, role_tab, Wb1, bb1, Wb2, bb2, Wf1, bf1, Wf2, bf2):
    raise NotImplementedError("write your pallas kernel here")


# speedup vs baseline: 15.0120x; 15.0120x over previous
import functools

import jax
import jax.numpy as jnp
from jax.experimental import pallas as pl


def _encoder_kernel(feats_ref, wb1t_ref, bb1_ref, w1c_ref, bz_ref, wf2t_ref,
                    bf2_ref, out_ref):
    f = feats_ref[...]
    fi = f.astype(jnp.int32)
    R = f.shape[0]

    behavior = f[:, 7:15]
    h = jnp.maximum(
        jnp.dot(behavior, wb1t_ref[...],
                preferred_element_type=jnp.float32) + bb1_ref[...], 0.0)

    lane = jax.lax.broadcasted_iota(jnp.int32, (R, 64), 1)
    mh = ((lane == fi[:, 0:1]).astype(jnp.float32)
          + (lane == fi[:, 1:2] + 9).astype(jnp.float32)
          + (lane == fi[:, 2:3] + 9).astype(jnp.float32)
          + (lane == fi[:, 3:4] + 9).astype(jnp.float32)
          + (lane == fi[:, 4:5] + 9).astype(jnp.float32)
          + (lane == fi[:, 5:6] + 18).astype(jnp.float32)
          + (lane == fi[:, 6:7] + 47).astype(jnp.float32))

    cat = jnp.concatenate([mh, h], axis=1)
    z = jnp.dot(cat, w1c_ref[...], preferred_element_type=jnp.float32)
    g = jnp.maximum(z + bz_ref[...], 0.0)
    out = jnp.dot(g, wf2t_ref[...], preferred_element_type=jnp.float32)
    out_ref[...] = out + bf2_ref[...]


@jax.jit
def kernel(player_features, hand_tab, suit_tab, bid_tab, role_tab,
           Wb1, bb1, Wb2, bb2, Wf1, bf1, Wf2, bf2):
    B, P, D = player_features.shape[0], player_features.shape[1], Wf1.shape[0]
    N = B * P
    feats = player_features.reshape(N, 15)

    Wf1T = Wf1.T
    t_hand = hand_tab @ Wf1T[0:16]
    t_suit = (0.25 * suit_tab) @ Wf1T[16:32]
    t_bid = bid_tab @ Wf1T[32:48]
    t_role = role_tab @ Wf1T[48:64]
    t_pad = jnp.zeros((13, D), dtype=jnp.float32)
    t_lut = jnp.concatenate([t_hand, t_suit, t_bid, t_role, t_pad], axis=0)

    w_beh = Wb2.T @ Wf1T[64:128]
    w1c = jnp.concatenate([t_lut, w_beh], axis=0)
    bz = (bf1 + bb2 @ Wf1T[64:128]).reshape(1, D)
    wb1t = Wb1.T
    wf2t = Wf2.T

    R = 4096
    grid = (N // R,)
    out = pl.pallas_call(
        _encoder_kernel,
        grid=grid,
        in_specs=[
            pl.BlockSpec((R, 15), lambda i: (i, 0)),
            pl.BlockSpec((8, 64), lambda i: (0, 0)),
            pl.BlockSpec((1, 64), lambda i: (0, 0)),
            pl.BlockSpec((128, 128), lambda i: (0, 0)),
            pl.BlockSpec((1, 128), lambda i: (0, 0)),
            pl.BlockSpec((128, 128), lambda i: (0, 0)),
            pl.BlockSpec((1, 128), lambda i: (0, 0)),
        ],
        out_specs=pl.BlockSpec((R, 128), lambda i: (i, 0)),
        out_shape=jax.ShapeDtypeStruct((N, D), jnp.float32),
    )(feats, wb1t, bb1.reshape(1, 64), w1c, bz, wf2t, bf2.reshape(1, D))
    return out.reshape(B, P, D)


# speedup vs baseline: 19.3882x; 1.2915x over previous
import functools

import jax
import jax.numpy as jnp
from jax.experimental import pallas as pl


def _encoder_kernel(feats_ref, s_ref, cst_ref, t_ref, wb1t_ref, bb1_ref,
                    wbeh_ref, bz_ref, wf2t_ref, bf2_ref, out_ref):
    f = feats_ref[...]
    R = f.shape[0]

    behavior = f[:, 7:15]
    h = jnp.maximum(
        jnp.dot(behavior, wb1t_ref[...],
                preferred_element_type=jnp.float32) + bb1_ref[...], 0.0)

    bc = jnp.dot(f[:, 0:8], s_ref[...], preferred_element_type=jnp.float32)
    mh = (bc == cst_ref[...]).astype(jnp.float32)

    z = (jnp.dot(mh, t_ref[...], preferred_element_type=jnp.float32)
         + jnp.dot(h, wbeh_ref[...], preferred_element_type=jnp.float32)
         + bz_ref[...])
    g = jnp.maximum(z, 0.0)
    out = jnp.dot(g, wf2t_ref[...], preferred_element_type=jnp.float32)
    out_ref[...] = out + bf2_ref[...]


@jax.jit
def kernel(player_features, hand_tab, suit_tab, bid_tab, role_tab,
           Wb1, bb1, Wb2, bb2, Wf1, bf1, Wf2, bf2):
    B, P, D = player_features.shape[0], player_features.shape[1], Wf1.shape[0]
    N = B * P
    feats = player_features.reshape(N, 15)

    Wf1T = Wf1.T
    t_hand = hand_tab @ Wf1T[0:16]
    t_suit = (0.25 * suit_tab) @ Wf1T[16:32]
    t_bid = bid_tab @ Wf1T[32:48]
    t_role = role_tab @ Wf1T[48:64]
    t_pad = jnp.zeros((13, D), dtype=jnp.float32)
    t_lut = jnp.concatenate([t_hand, t_suit, t_bid, t_role, t_pad], axis=0)
    t512 = jnp.concatenate([jnp.tile(t_lut, (7, 1)),
                            jnp.zeros((64, D), jnp.float32)], axis=0)

    eye8 = jnp.eye(8, dtype=jnp.float32).at[7, 7].set(0.0)
    s_mat = jnp.repeat(eye8, 64, axis=1)
    offs = jnp.array([0, 9, 9, 9, 9, 18, 47], jnp.float32)
    lane = jnp.arange(64, dtype=jnp.float32)
    cst7 = (lane[None, :] - offs[:, None]).reshape(1, 448)
    cst = jnp.concatenate(
        [cst7, jnp.full((1, 64), jnp.nan, jnp.float32)], axis=1)

    w_beh = Wb2.T @ Wf1T[64:128]
    bz = (bf1 + bb2 @ Wf1T[64:128]).reshape(1, D)
    wb1t = Wb1.T
    wf2t = Wf2.T

    R = 2048
    grid = (N // R,)
    out = pl.pallas_call(
        _encoder_kernel,
        grid=grid,
        in_specs=[
            pl.BlockSpec((R, 15), lambda i: (i, 0)),
            pl.BlockSpec((8, 512), lambda i: (0, 0)),
            pl.BlockSpec((1, 512), lambda i: (0, 0)),
            pl.BlockSpec((512, 128), lambda i: (0, 0)),
            pl.BlockSpec((8, 64), lambda i: (0, 0)),
            pl.BlockSpec((1, 64), lambda i: (0, 0)),
            pl.BlockSpec((64, 128), lambda i: (0, 0)),
            pl.BlockSpec((1, 128), lambda i: (0, 0)),
            pl.BlockSpec((128, 128), lambda i: (0, 0)),
            pl.BlockSpec((1, 128), lambda i: (0, 0)),
        ],
        out_specs=pl.BlockSpec((R, 128), lambda i: (i, 0)),
        out_shape=jax.ShapeDtypeStruct((N, D), jnp.float32),
    )(feats, s_mat, cst, t512, wb1t, bb1.reshape(1, 64), w_beh, bz, wf2t,
      bf2.reshape(1, D))
    return out.reshape(B, P, D)


# speedup vs baseline: 20.1531x; 1.0394x over previous
import functools

import jax
import jax.numpy as jnp
from jax.experimental import pallas as pl


def _encoder_kernel(feats_ref, s_ref, cst_ref, t_ref, wb1t_ref, bb1_ref,
                    wbeh_ref, bz_ref, wf2t_ref, bf2_ref, out_ref):
    f = feats_ref[...]
    R = f.shape[0]

    behavior = f[:, 7:15]
    h = jnp.maximum(
        jnp.dot(behavior, wb1t_ref[...],
                preferred_element_type=jnp.float32) + bb1_ref[...], 0.0)

    bc = jnp.dot(f[:, 0:8], s_ref[...], preferred_element_type=jnp.float32)
    mh = (bc == cst_ref[...]).astype(jnp.float32)

    z = (jnp.dot(mh, t_ref[...], preferred_element_type=jnp.float32)
         + jnp.dot(h, wbeh_ref[...], preferred_element_type=jnp.float32)
         + bz_ref[...])
    g = jnp.maximum(z, 0.0)
    out = jnp.dot(g, wf2t_ref[...], preferred_element_type=jnp.float32)
    out_ref[...] = out + bf2_ref[...]


@jax.jit
def kernel(player_features, hand_tab, suit_tab, bid_tab, role_tab,
           Wb1, bb1, Wb2, bb2, Wf1, bf1, Wf2, bf2):
    B, P, D = player_features.shape[0], player_features.shape[1], Wf1.shape[0]
    N = B * P
    feats = player_features.reshape(N, 15)

    Wf1T = Wf1.T
    t_hand = hand_tab @ Wf1T[0:16]
    t_suit = (0.25 * suit_tab) @ Wf1T[16:32]
    t_bid = bid_tab @ Wf1T[32:48]
    t_role = role_tab @ Wf1T[48:64]
    def pad_rows(t, n):
        return jnp.concatenate(
            [t, jnp.zeros((n - t.shape[0], D), jnp.float32)], axis=0)
    t128 = jnp.concatenate(
        [pad_rows(t_hand, 16), pad_rows(t_suit, 16), pad_rows(t_suit, 16),
         pad_rows(t_suit, 16), pad_rows(t_suit, 16), pad_rows(t_bid, 32),
         pad_rows(t_role, 16)], axis=0)

    seg_starts = jnp.array([0, 16, 32, 48, 64, 80, 112], jnp.int32)
    seg_widths = jnp.array([16, 16, 16, 16, 16, 32, 16], jnp.int32)
    lane = jnp.arange(128, dtype=jnp.int32)
    in_seg = ((lane[None, :] >= seg_starts[:, None])
              & (lane[None, :] < (seg_starts + seg_widths)[:, None]))
    s_mat = jnp.concatenate(
        [in_seg.astype(jnp.float32),
         jnp.zeros((1, 128), jnp.float32)], axis=0)
    seg_of_lane = (lane[None, :] >= seg_starts[:, None]).astype(
        jnp.int32).sum(axis=0) - 1
    cst = (lane - seg_starts[seg_of_lane]).astype(
        jnp.float32).reshape(1, 128)

    w_beh = Wb2.T @ Wf1T[64:128]
    bz = (bf1 + bb2 @ Wf1T[64:128]).reshape(1, D)
    wb1t = Wb1.T
    wf2t = Wf2.T

    R = 2048
    grid = (N // R,)
    out = pl.pallas_call(
        _encoder_kernel,
        grid=grid,
        in_specs=[
            pl.BlockSpec((R, 15), lambda i: (i, 0)),
            pl.BlockSpec((8, 128), lambda i: (0, 0)),
            pl.BlockSpec((1, 128), lambda i: (0, 0)),
            pl.BlockSpec((128, 128), lambda i: (0, 0)),
            pl.BlockSpec((8, 64), lambda i: (0, 0)),
            pl.BlockSpec((1, 64), lambda i: (0, 0)),
            pl.BlockSpec((64, 128), lambda i: (0, 0)),
            pl.BlockSpec((1, 128), lambda i: (0, 0)),
            pl.BlockSpec((128, 128), lambda i: (0, 0)),
            pl.BlockSpec((1, 128), lambda i: (0, 0)),
        ],
        out_specs=pl.BlockSpec((R, 128), lambda i: (i, 0)),
        out_shape=jax.ShapeDtypeStruct((N, D), jnp.float32),
    )(feats, s_mat, cst, t128, wb1t, bb1.reshape(1, 64), w_beh, bz, wf2t,
      bf2.reshape(1, D))
    return out.reshape(B, P, D)


# speedup vs baseline: 23.2815x; 1.1552x over previous
import functools

import jax
import jax.numpy as jnp
from jax.experimental import pallas as pl


def _encoder_kernel(feats_ref, s_ref, cst_ref, t_ref, wb1t_ref, bb1_ref,
                    wbeh_ref, bz_ref, wf2t_ref, bf2_ref, out_ref):
    f = feats_ref[...]
    R = f.shape[0]

    behavior = f[:, 7:15]
    h = jnp.maximum(
        jnp.dot(behavior, wb1t_ref[...],
                preferred_element_type=jnp.float32) + bb1_ref[...], 0.0)

    bc = jnp.dot(f[:, 0:8], s_ref[...], preferred_element_type=jnp.float32)
    mh = (bc == cst_ref[...]).astype(jnp.float32)

    z = (jnp.dot(mh, t_ref[...], preferred_element_type=jnp.float32)
         + jnp.dot(h, wbeh_ref[...], preferred_element_type=jnp.float32)
         + bz_ref[...])
    g = jnp.maximum(z, 0.0)
    out = jnp.dot(g, wf2t_ref[...], preferred_element_type=jnp.float32)
    out_ref[...] = out + bf2_ref[...]


@jax.jit
def kernel(player_features, hand_tab, suit_tab, bid_tab, role_tab,
           Wb1, bb1, Wb2, bb2, Wf1, bf1, Wf2, bf2):
    B, P, D = player_features.shape[0], player_features.shape[1], Wf1.shape[0]
    N = B * P
    feats = player_features.reshape(N, 15)

    Wf1T = Wf1.T
    t_hand = hand_tab @ Wf1T[0:16]
    t_suit = (0.25 * suit_tab) @ Wf1T[16:32]
    t_bid = bid_tab @ Wf1T[32:48]
    t_role = role_tab @ Wf1T[48:64]
    def pad_rows(t, n):
        return jnp.concatenate(
            [t, jnp.zeros((n - t.shape[0], D), jnp.float32)], axis=0)
    t128 = jnp.concatenate(
        [pad_rows(t_hand, 16), pad_rows(t_suit, 16), pad_rows(t_suit, 16),
         pad_rows(t_suit, 16), pad_rows(t_suit, 16), pad_rows(t_bid, 32),
         pad_rows(t_role, 16)], axis=0)

    seg_starts = jnp.array([0, 16, 32, 48, 64, 80, 112], jnp.int32)
    seg_widths = jnp.array([16, 16, 16, 16, 16, 32, 16], jnp.int32)
    lane = jnp.arange(128, dtype=jnp.int32)
    in_seg = ((lane[None, :] >= seg_starts[:, None])
              & (lane[None, :] < (seg_starts + seg_widths)[:, None]))
    s_mat = jnp.concatenate(
        [in_seg.astype(jnp.float32),
         jnp.zeros((1, 128), jnp.float32)], axis=0)
    seg_of_lane = (lane[None, :] >= seg_starts[:, None]).astype(
        jnp.int32).sum(axis=0) - 1
    cst = (lane - seg_starts[seg_of_lane]).astype(
        jnp.float32).reshape(1, 128)

    w_beh = Wb2.T @ Wf1T[64:128]
    bz = (bf1 + bb2 @ Wf1T[64:128]).reshape(1, D)
    wb1t = Wb1.T
    wf2t = Wf2.T

    R = 8192
    grid = (N // R,)
    out = pl.pallas_call(
        _encoder_kernel,
        grid=grid,
        in_specs=[
            pl.BlockSpec((R, 15), lambda i: (i, 0)),
            pl.BlockSpec((8, 128), lambda i: (0, 0)),
            pl.BlockSpec((1, 128), lambda i: (0, 0)),
            pl.BlockSpec((128, 128), lambda i: (0, 0)),
            pl.BlockSpec((8, 64), lambda i: (0, 0)),
            pl.BlockSpec((1, 64), lambda i: (0, 0)),
            pl.BlockSpec((64, 128), lambda i: (0, 0)),
            pl.BlockSpec((1, 128), lambda i: (0, 0)),
            pl.BlockSpec((128, 128), lambda i: (0, 0)),
            pl.BlockSpec((1, 128), lambda i: (0, 0)),
        ],
        out_specs=pl.BlockSpec((R, 128), lambda i: (i, 0)),
        out_shape=jax.ShapeDtypeStruct((N, D), jnp.float32),
    )(feats, s_mat, cst, t128, wb1t, bb1.reshape(1, 64), w_beh, bz, wf2t,
      bf2.reshape(1, D))
    return out.reshape(B, P, D)
